# baseline (speedup 1.0000x reference)
import functools

import jax
import jax.numpy as jnp
from jax import lax
from jax.experimental import pallas as pl
from jax.experimental.pallas import tpu as pltpu
from jax.experimental.pallas import tpu_sc as plsc

N = 10000
E = 320000
B = 128
T = 5
LR = 0.01
N_SENSORY = 2048

NC = 2
NSUB = 16
NW = NC * NSUB
CHUNK = 64
NCHUNK = 158
EW = NCHUNK * CHUNK
EPAD = EW * NW
SUB_STRIDE = 624
SUB_ROWS = 640

_mesh = plsc.VectorSubcoreMesh(core_axis_name="c", subcore_axis_name="s")


def _sc_pass_body(tab_hbm, gidx_hbm, sidx_hbm, w_hbm, out_hbm,
                  gidx_v, sidx_v, w_v, rows_v0, rows_v1, y_sh,
                  gsem0, gsem1, ssem0, ssem1):
    c = lax.axis_index("c")
    s = lax.axis_index("s")
    g = c * NSUB + s

    pltpu.async_copy(gidx_hbm.at[g], gidx_v, gsem0)
    pltpu.async_copy(sidx_hbm.at[g], sidx_v, gsem1)
    pltpu.async_copy(w_hbm.at[g], w_v, ssem0)

    def _zrow(j, carry):
        for r in range(B // 16):
            rows_v0[j, pl.ds(r * 16, 16)] = jnp.zeros((16,), jnp.float32)
        return carry
    lax.fori_loop(0, CHUNK, _zrow, 0)
    base = s * SUB_STRIDE
    for k in range(SUB_ROWS // CHUNK):
        pltpu.async_copy(rows_v0, y_sh.at[pl.ds(base + k * CHUNK, CHUNK)],
                         ssem1)
    for k in range(SUB_ROWS // CHUNK):
        pltpu.make_async_copy(rows_v0, y_sh.at[pl.ds(base, CHUNK)],
                              ssem1).wait()
    pltpu.make_async_copy(gidx_hbm.at[g], gidx_v, gsem0).wait()
    pltpu.make_async_copy(sidx_hbm.at[g], sidx_v, gsem1).wait()
    pltpu.make_async_copy(w_hbm.at[g], w_v, ssem0).wait()
    plsc.subcore_barrier()

    bufs = (rows_v0, rows_v1)
    gsems = (gsem0, gsem1)
    ssems = (ssem0, ssem1)

    pltpu.async_copy(tab_hbm.at[gidx_v.at[0]], rows_v0, gsem0)

    def _pair(pi, carry):
        for b in range(2):
            cc = pi * 2 + b
            nb = 1 - b
            buf = bufs[b]
            nbuf = bufs[nb]
            pltpu.make_async_copy(tab_hbm.at[pl.ds(0, CHUNK)], buf,
                                  gsems[b]).wait()

            @pl.when(cc >= 1)
            def _():
                pltpu.make_async_copy(tab_hbm.at[pl.ds(0, CHUNK)], nbuf,
                                      ssems[nb]).wait()

            @pl.when(cc + 1 < NCHUNK)
            def _():
                pltpu.async_copy(tab_hbm.at[gidx_v.at[cc + 1]], nbuf,
                                 gsems[nb])

            for j2 in range(CHUNK // 16):
                wvec = w_v[cc, pl.ds(j2 * 16, 16)]
                for l in range(16):
                    wj = wvec[l]
                    e = j2 * 16 + l
                    for r in range(B // 16):
                        buf[e, pl.ds(r * 16, 16)] = buf[e, pl.ds(r * 16, 16)] * wj

            pltpu.async_copy(buf, y_sh.at[sidx_v.at[cc]], ssems[b], add=True)
        return carry
    lax.fori_loop(0, NCHUNK // 2, _pair, 0)
    pltpu.make_async_copy(tab_hbm.at[pl.ds(0, CHUNK)],
                          bufs[(NCHUNK - 1) % 2],
                          ssems[(NCHUNK - 1) % 2]).wait()
    plsc.subcore_barrier()

    pltpu.sync_copy(y_sh.at[pl.ds(base, SUB_ROWS)],
                    out_hbm.at[c, pl.ds(base, SUB_ROWS)])


_sc_pass = functools.partial(
    pl.kernel,
    out_type=jax.ShapeDtypeStruct((NC, N, B), jnp.float32),
    mesh=_mesh,
    scratch_types=[
        pltpu.VMEM((NCHUNK, CHUNK), jnp.int32),
        pltpu.VMEM((NCHUNK, CHUNK), jnp.int32),
        pltpu.VMEM((NCHUNK, CHUNK), jnp.float32),
        pltpu.VMEM((CHUNK, B), jnp.float32),
        pltpu.VMEM((CHUNK, B), jnp.float32),
        pltpu.VMEM_SHARED((N, B), jnp.float32),
        pltpu.SemaphoreType.DMA,
        pltpu.SemaphoreType.DMA,
        pltpu.SemaphoreType.DMA,
        pltpu.SemaphoreType.DMA,
    ],
    compiler_params=pltpu.CompilerParams(use_tc_tiling_on_sc=False),
)(_sc_pass_body)


_RB = 1000
_GRID = N // _RB
_full = pl.BlockSpec((_RB, B), lambda i: (i, 0))
_parts = pl.BlockSpec((NC, _RB, B), lambda i: (0, i, 0))


def _act_body(v_ref, a_ref):
    a_ref[...] = jnp.tanh(v_ref[...])


_act_call = pl.pallas_call(
    _act_body, grid=(_GRID,),
    in_specs=[_full], out_specs=_full,
    out_shape=jax.ShapeDtypeStruct((N, B), jnp.float32))


def _err_body(v_ref, p_ref, e_ref):
    e_ref[...] = v_ref[...] - p_ref[0] - p_ref[1]


_err_call = pl.pallas_call(
    _err_body, grid=(_GRID,),
    in_specs=[_full, _parts], out_specs=_full,
    out_shape=jax.ShapeDtypeStruct((N, B), jnp.float32))


def _upd_body(v_ref, a_ref, e_ref, b_ref, vo_ref, ao_ref):
    act = a_ref[...]
    back = (b_ref[0] + b_ref[1]) * (1.0 - act * act)
    grad = e_ref[...] - back
    rows = pl.program_id(0) * _RB + lax.broadcasted_iota(jnp.int32, (_RB, B), 0)
    mask = (rows >= N_SENSORY).astype(jnp.float32)
    vn = v_ref[...] - LR * mask * grad
    vo_ref[...] = vn
    ao_ref[...] = jnp.tanh(vn)


_upd_call = pl.pallas_call(
    _upd_body, grid=(_GRID,),
    in_specs=[_full, _full, _full, _parts],
    out_specs=[_full, _full],
    out_shape=[jax.ShapeDtypeStruct((N, B), jnp.float32),
               jax.ShapeDtypeStruct((N, B), jnp.float32)])


def kernel(x, edge_index, weights):
    src = edge_index[0]
    dst = edge_index[1]
    pad = EPAD - E
    srcp = jnp.pad(src, (0, pad)).reshape(NW, NCHUNK, CHUNK)
    dstp = jnp.pad(dst, (0, pad)).reshape(NW, NCHUNK, CHUNK)
    wp = jnp.pad(weights, (0, pad)).reshape(NW, NCHUNK, CHUNK)

    values = x
    act = _act_call(values)
    for _ in range(T):
        pred = _sc_pass(act, srcp, dstp, wp)
        err = _err_call(values, pred)
        back = _sc_pass(err, dstp, srcp, wp)
        values, act = _upd_call(values, act, err, back)
    return values
